# baseline (speedup 1.0000x reference)
import functools
import math

import jax
import jax.numpy as jnp
import numpy as np
from jax.experimental import pallas as pl
from jax.experimental.pallas import tpu as pltpu
from jax.experimental.pallas import tpu_sc as plsc

B = 8
M = 256
EMBD = 128
K3D = 128
MAX_Z = 101
OFF = 128

_R = 4096
_A = (2 * 3.14159) ** 0.5
_INV_SQRT2 = 1.0 / math.sqrt(2.0)


_TWO_PI_PARTS = (6.28125, 0.0019353071693331003)
_INV_TWO_PI = float(np.float32(1.0 / (2.0 * np.pi)))
_SIN_COEFS = (2.1401396767539715e-06, -0.00019249443151001314,
              0.008307955164852027, -0.16662189927828033,
              0.9999778011834951)
_HALF_PI_SQ = float(np.float32((np.pi / 2.0) ** 2))


def _sincos_premul(phase, kf):
    r = phase
    for p in _TWO_PI_PARTS:
        r = r - kf * p
    s = r * r
    pol = _SIN_COEFS[0]
    for c in _SIN_COEFS[1:]:
        pol = pol * s + c
    sin_v = r * pol
    w = jnp.maximum(1.0 - sin_v * sin_v, 1e-30)
    cmag = w * jax.lax.rsqrt(w)
    cos_v = jnp.where(s < _HALF_PI_SQ, cmag, -cmag)
    return sin_v, cos_v


def _acos(z):
    return jnp.arctan2(jnp.sqrt(jnp.maximum(1.0 - z * z, 0.0)), z)


def _geom_kernel(pos_col_ref, pos_row_ref, d_ref, az_ref, pol_ref):
    pc = pos_col_ref[0]
    pr = pos_row_ref[0]
    dx = pr[0:1, :] - pc[:, 0:1]
    dy = pr[1:2, :] - pc[:, 1:2]
    dz = pr[2:3, :] - pc[:, 2:3]
    s = dx * dx + dy * dy + dz * dz
    d_ref[0] = jnp.sqrt(s + 1e-12)
    az_ref[0] = jnp.arctan2(dy, dx)
    ndz = dz / (jnp.sqrt(s) + 1e-5)
    pol_ref[0] = _acos(jnp.clip(ndz, -1.0, 1.0))


def _pair_kernel(geo_ref, mb_ref, means_ref, stds_ref,
                 l1w_ref, l1b_ref, w2_ref, b2_ref, fr_ref, out_ref):
    geo = geo_ref[...]
    d = geo[:, 0:1]
    az = geo[:, 1:2]
    po = geo[:, 2:3]
    mul = mb_ref[0, 0]
    bias = mb_ref[0, 1]
    std = jnp.abs(stds_ref[...]) + 0.01
    inv_std = 1.0 / std
    _KE = 0.8493218002880191
    ac = (mul * _KE) * inv_std
    cc = (bias - means_ref[...]) * inv_std * _KE
    c2 = -jnp.log2(_A * std)
    arg = d * ac + cc
    gk = jnp.exp2(c2 - arg * arg)
    hid2 = jnp.dot(gk.astype(jnp.bfloat16), l1w_ref[...],
                   preferred_element_type=jnp.float32)
    hid2 = hid2 + l1b_ref[...]
    h2 = _INV_SQRT2 * hid2
    hid = h2 * jax.lax.erf(hid2) + h2
    fr = fr_ref[...]
    azpo = jnp.concatenate(
        [jnp.broadcast_to(az, (az.shape[0], 64)),
         jnp.broadcast_to(po, (po.shape[0], 64))], axis=1)
    ph = azpo * fr
    kf = jnp.floor(ph * _INV_TWO_PI + 0.5)
    sin_c, cos_c = _sincos_premul(ph, kf)
    feats = jnp.concatenate(
        [hid.astype(jnp.bfloat16),
         sin_c.astype(jnp.bfloat16), cos_c.astype(jnp.bfloat16)],
        axis=1)
    e = jnp.dot(feats, w2_ref[...], preferred_element_type=jnp.float32)
    out_ref[...] = e + b2_ref[...]


def _h_table_kernel(emb_ref, ec_ref, cfgwt_ref, cfgb_ref, mult_ref, chg_ref,
                    multtab_ref, chgtab_ref, t2_ref):
    f32 = jnp.float32
    elec = jnp.dot(ec_ref[...], cfgwt_ref[...],
                   preferred_element_type=f32) + cfgb_ref[...]
    t = emb_ref[...] + elec
    lane = jax.lax.broadcasted_iota(jnp.int32, (1, 128), 1)
    moh = (mult_ref[...] == lane).astype(f32)
    coh = ((chg_ref[...] + OFF // 2) == lane).astype(f32)
    g = jnp.dot(moh, multtab_ref[...], preferred_element_type=f32)
    g = g + jnp.dot(coh, chgtab_ref[...], preferred_element_type=f32)
    row = jax.lax.broadcasted_iota(jnp.int32, (128, 1), 0)
    boh = (row == (MAX_Z + 1) +
           jax.lax.broadcasted_iota(jnp.int32, (1, B), 1)).astype(f32)
    grows = jnp.dot(boh, g, preferred_element_type=f32)
    is_cls = ((row >= MAX_Z + 1) & (row < MAX_Z + 1 + B)).astype(f32)
    emb101 = emb_ref[MAX_Z:MAX_Z + 1, :]
    t2_ref[...] = t + is_cls * (emb101 + grows - t)


def _sc_gather(table, idx):
    n = idx.shape[0]
    win = 128
    mesh = plsc.VectorSubcoreMesh(core_axis_name="c", subcore_axis_name="s")

    @functools.partial(
        pl.kernel,
        out_type=jax.ShapeDtypeStruct((n, table.shape[1]), table.dtype),
        mesh=mesh)
    def k(x_hbm, i_hbm, o_hbm):
        def body(i_vmem, o_vmem):
            pltpu.sync_copy(x_hbm.at[i_vmem.at[0]], o_vmem)

        pltpu.emit_pipeline(
            body,
            grid=(n // win,),
            in_specs=[pl.BlockSpec((1, win), index_map=lambda i: (0, i))],
            out_specs=[pl.BlockSpec((win, table.shape[1]),
                                    index_map=lambda i: (i, 0))],
            core_axis_name="s",
            dimension_semantics=(pltpu.PARALLEL,),
        )(i_hbm, o_hbm)

    return k(table, idx.reshape(1, n))


def kernel(positions, atomic_numbers, mask, multiplicity, charge, emb_table,
           electron_config, cfg_W, cfg_b, mult_table, charge_table, means,
           stds, mul_w, bias_w, l1_W, l1_b, l2_W, l2_b, freqs_az, freqs_po,
           proj_W, proj_b):
    f32 = jnp.float32
    pos = jnp.concatenate([jnp.zeros_like(positions[:, :1]), positions], 1)
    az_full = jnp.concatenate(
        [jnp.full_like(atomic_numbers[:, :1], MAX_Z), atomic_numbers], 1)
    msk = jnp.concatenate([jnp.ones_like(mask[:, :1]), mask], 1)

    pos_row = jnp.transpose(pos, (0, 2, 1))
    d, azm, pol = pl.pallas_call(
        _geom_kernel,
        grid=(B,),
        in_specs=[
            pl.BlockSpec((1, M, 3), lambda b: (b, 0, 0)),
            pl.BlockSpec((1, 3, M), lambda b: (b, 0, 0)),
        ],
        out_specs=[pl.BlockSpec((1, M, M), lambda b: (b, 0, 0))] * 3,
        out_shape=[jax.ShapeDtypeStruct((B, M, M), f32)] * 3,
    )(pos, pos_row)

    nrows = B * M * M
    grid = nrows // _R
    geo = jnp.stack([d, azm, pol], axis=-1).reshape(nrows, 3)
    mb = jnp.stack([mul_w[0, 0], bias_w[0, 0]]).reshape(1, 2)
    col = pl.BlockSpec((_R, 3), lambda g: (g, 0))
    full = lambda shape: pl.BlockSpec(shape, lambda g: (0,) * len(shape))
    bf16 = jnp.bfloat16
    pt = proj_W.T
    w2 = jnp.concatenate(
        [l2_W.T, pt[0:64], pt[128:192], pt[64:128], pt[192:256]],
        axis=0).astype(bf16)
    b2 = (l2_b + proj_b).reshape(1, EMBD)
    e_flat = pl.pallas_call(
        _pair_kernel,
        grid=(grid,),
        in_specs=[
            col,
            full((1, 2)),
            full((1, K3D)), full((1, K3D)),
            full((K3D, K3D)), full((1, K3D)),
            full((K3D + 256, EMBD)), full((1, EMBD)),
            full((1, 128)),
        ],
        out_specs=pl.BlockSpec((_R, EMBD), lambda g: (g, 0)),
        out_shape=jax.ShapeDtypeStruct((nrows, EMBD), f32),
        compiler_params=pltpu.CompilerParams(
            dimension_semantics=("parallel",)),
    )(geo, mb, means.reshape(1, K3D), stds.reshape(1, K3D),
      (l1_W.T * _INV_SQRT2).astype(bf16),
      (l1_b * _INV_SQRT2).reshape(1, K3D), w2, b2,
      jnp.concatenate([freqs_az, freqs_po]).reshape(1, 128))
    e = e_flat.reshape(B, M, M, EMBD)

    pad = 128 - (MAX_Z + 1)
    emb_pad = jnp.pad(emb_table, ((0, pad), (0, 0)))
    ec_pad = jnp.pad(electron_config, ((0, pad), (0, 0)))
    t2 = pl.pallas_call(
        _h_table_kernel,
        grid=(1,),
        in_specs=[
            full((128, EMBD)), full((128, 20)), full((20, EMBD)),
            full((1, EMBD)), full((B, 1)), full((B, 1)),
            full((OFF, EMBD)), full((OFF, EMBD)),
        ],
        out_specs=pl.BlockSpec((128, EMBD), lambda g: (0, 0)),
        out_shape=jax.ShapeDtypeStruct((128, EMBD), f32),
    )(emb_pad, ec_pad, cfg_W.T, cfg_b.reshape(1, EMBD), multiplicity,
      charge, mult_table, charge_table)
    cls_ids = MAX_Z + 1 + jnp.arange(B, dtype=jnp.int32)[:, None]
    az_idx = jnp.concatenate([cls_ids, atomic_numbers], axis=1)
    h = _sc_gather(t2, az_idx.reshape(B * M)).reshape(B, M, EMBD)
    return (h, e, msk)


# speedup vs baseline: 1.0009x; 1.0009x over previous
import functools
import math

import jax
import jax.numpy as jnp
import numpy as np
from jax.experimental import pallas as pl
from jax.experimental.pallas import tpu as pltpu
from jax.experimental.pallas import tpu_sc as plsc

B = 8
M = 256
EMBD = 128
K3D = 128
MAX_Z = 101
OFF = 128

_R = 4096
_A = (2 * 3.14159) ** 0.5
_INV_SQRT2 = 1.0 / math.sqrt(2.0)


_TWO_PI_PARTS = (6.28125, 0.0019353071693331003)
_INV_TWO_PI = float(np.float32(1.0 / (2.0 * np.pi)))
_SIN_COEFS = (2.1401396767539715e-06, -0.00019249443151001314,
              0.008307955164852027, -0.16662189927828033,
              0.9999778011834951)
_HALF_PI_SQ = float(np.float32((np.pi / 2.0) ** 2))


def _sincos_premul(phase, kf):
    r = phase
    for p in _TWO_PI_PARTS:
        r = r - kf * p
    s = r * r
    pol = _SIN_COEFS[0]
    for c in _SIN_COEFS[1:]:
        pol = pol * s + c
    sin_v = r * pol
    w = jnp.maximum(1.0 - sin_v * sin_v, 1e-30)
    cmag = w * jax.lax.rsqrt(w)
    cos_v = jnp.where(s < _HALF_PI_SQ, cmag, -cmag)
    return sin_v, cos_v


def _acos(z):
    return jnp.arctan2(jnp.sqrt(jnp.maximum(1.0 - z * z, 0.0)), z)


def _geom_kernel(pos_col_ref, pos_row_ref, d_ref, az_ref, pol_ref):
    pc = pos_col_ref[0]
    pr = pos_row_ref[0]
    dx = pr[0:1, :] - pc[:, 0:1]
    dy = pr[1:2, :] - pc[:, 1:2]
    dz = pr[2:3, :] - pc[:, 2:3]
    s = dx * dx + dy * dy + dz * dz
    d_ref[0] = jnp.sqrt(s + 1e-12)
    az_ref[0] = jnp.arctan2(dy, dx)
    ndz = dz / (jnp.sqrt(s) + 1e-5)
    pol_ref[0] = _acos(jnp.clip(ndz, -1.0, 1.0))


def _pair_kernel(geo_ref, mb_ref, means_ref, stds_ref,
                 l1w_ref, l1b_ref, w2_ref, b2_ref, fr_ref, out_ref):
    geo = geo_ref[...]
    d = geo[:, 0:1]
    az = geo[:, 1:2]
    po = geo[:, 2:3]
    mul = mb_ref[0, 0]
    bias = mb_ref[0, 1]
    std = jnp.abs(stds_ref[...]) + 0.01
    inv_std = 1.0 / std
    _KE = 0.8493218002880191
    ac = (mul * _KE) * inv_std
    cc = (bias - means_ref[...]) * inv_std * _KE
    c2 = -jnp.log2(_A * std)
    arg = d * ac + cc
    gk = jnp.exp2(c2 - arg * arg)
    hid2 = jnp.dot(gk.astype(jnp.bfloat16), l1w_ref[...],
                   preferred_element_type=jnp.float32)
    hid2 = hid2 + l1b_ref[...]
    h2 = _INV_SQRT2 * hid2
    hid = h2 * jax.lax.erf(hid2) + h2
    fr = fr_ref[...]
    azpo = jnp.concatenate(
        [jnp.broadcast_to(az, (az.shape[0], 64)),
         jnp.broadcast_to(po, (po.shape[0], 64))], axis=1)
    ph = azpo * fr
    kf = jnp.floor(ph * _INV_TWO_PI + 0.5)
    sin_c, cos_c = _sincos_premul(ph, kf)
    feats = jnp.concatenate(
        [hid.astype(jnp.bfloat16),
         sin_c.astype(jnp.bfloat16), cos_c.astype(jnp.bfloat16)],
        axis=1)
    e = jnp.dot(feats, w2_ref[...], preferred_element_type=jnp.float32)
    out_ref[...] = e + b2_ref[...]


def _h_table_kernel(emb_ref, ec_ref, cfgwt_ref, cfgb_ref, mult_ref, chg_ref,
                    multtab_ref, chgtab_ref, t2_ref):
    f32 = jnp.float32
    elec = jnp.dot(ec_ref[...], cfgwt_ref[...],
                   preferred_element_type=f32) + cfgb_ref[...]
    t = emb_ref[...] + elec
    lane = jax.lax.broadcasted_iota(jnp.int32, (1, 128), 1)
    moh = (mult_ref[...] == lane).astype(f32)
    coh = ((chg_ref[...] + OFF // 2) == lane).astype(f32)
    g = jnp.dot(moh, multtab_ref[...], preferred_element_type=f32)
    g = g + jnp.dot(coh, chgtab_ref[...], preferred_element_type=f32)
    row = jax.lax.broadcasted_iota(jnp.int32, (128, 1), 0)
    boh = (row == (MAX_Z + 1) +
           jax.lax.broadcasted_iota(jnp.int32, (1, B), 1)).astype(f32)
    grows = jnp.dot(boh, g, preferred_element_type=f32)
    is_cls = ((row >= MAX_Z + 1) & (row < MAX_Z + 1 + B)).astype(f32)
    emb101 = emb_ref[MAX_Z:MAX_Z + 1, :]
    t2_ref[...] = t + is_cls * (emb101 + grows - t)


def _sc_gather(table, idx):
    n = idx.shape[0]
    win = 128
    mesh = plsc.VectorSubcoreMesh(core_axis_name="c", subcore_axis_name="s")

    @functools.partial(
        pl.kernel,
        out_type=jax.ShapeDtypeStruct((n, table.shape[1]), table.dtype),
        mesh=mesh)
    def k(x_hbm, i_hbm, o_hbm):
        def body(i_vmem, o_vmem):
            pltpu.sync_copy(x_hbm.at[i_vmem.at[0]], o_vmem)

        pltpu.emit_pipeline(
            body,
            grid=(n // win,),
            in_specs=[pl.BlockSpec((1, win), index_map=lambda i: (0, i))],
            out_specs=[pl.BlockSpec((win, table.shape[1]),
                                    index_map=lambda i: (i, 0))],
            core_axis_name="s",
            dimension_semantics=(pltpu.PARALLEL,),
        )(i_hbm, o_hbm)

    return k(table, idx.reshape(1, n))


def kernel(positions, atomic_numbers, mask, multiplicity, charge, emb_table,
           electron_config, cfg_W, cfg_b, mult_table, charge_table, means,
           stds, mul_w, bias_w, l1_W, l1_b, l2_W, l2_b, freqs_az, freqs_po,
           proj_W, proj_b):
    f32 = jnp.float32
    pos = jnp.concatenate([jnp.zeros_like(positions[:, :1]), positions], 1)
    az_full = jnp.concatenate(
        [jnp.full_like(atomic_numbers[:, :1], MAX_Z), atomic_numbers], 1)
    msk = jnp.concatenate([jnp.ones_like(mask[:, :1]), mask], 1)

    pos_row = jnp.transpose(pos, (0, 2, 1))
    d, azm, pol = pl.pallas_call(
        _geom_kernel,
        grid=(B,),
        in_specs=[
            pl.BlockSpec((1, M, 3), lambda b: (b, 0, 0)),
            pl.BlockSpec((1, 3, M), lambda b: (b, 0, 0)),
        ],
        out_specs=[pl.BlockSpec((1, M, M), lambda b: (b, 0, 0))] * 3,
        out_shape=[jax.ShapeDtypeStruct((B, M, M), f32)] * 3,
    )(pos, pos_row)

    nrows = B * M * M
    grid = nrows // _R
    geo = jnp.stack([d, azm, pol], axis=-1).reshape(nrows, 3)
    mb = jnp.stack([mul_w[0, 0], bias_w[0, 0]]).reshape(1, 2)
    col = pl.BlockSpec((_R, 3), lambda g: (g, 0))
    full = lambda shape: pl.BlockSpec(shape, lambda g: (0,) * len(shape))
    bf16 = jnp.bfloat16
    pt = proj_W.T
    w2 = jnp.concatenate(
        [l2_W.T, pt[0:64], pt[128:192], pt[64:128], pt[192:256]],
        axis=0).astype(bf16)
    b2 = (l2_b + proj_b).reshape(1, EMBD)
    e_flat = pl.pallas_call(
        _pair_kernel,
        grid=(grid,),
        in_specs=[
            col,
            full((1, 2)),
            full((1, K3D)), full((1, K3D)),
            full((K3D, K3D)), full((1, K3D)),
            full((K3D + 256, EMBD)), full((1, EMBD)),
            full((1, 128)),
        ],
        out_specs=pl.BlockSpec((_R, EMBD), lambda g: (g, 0)),
        out_shape=jax.ShapeDtypeStruct((nrows, EMBD), f32),
        compiler_params=pltpu.CompilerParams(
            dimension_semantics=("arbitrary",)),
    )(geo, mb, means.reshape(1, K3D), stds.reshape(1, K3D),
      (l1_W.T * _INV_SQRT2).astype(bf16),
      (l1_b * _INV_SQRT2).reshape(1, K3D), w2, b2,
      jnp.concatenate([freqs_az, freqs_po]).reshape(1, 128))
    e = e_flat.reshape(B, M, M, EMBD)

    pad = 128 - (MAX_Z + 1)
    emb_pad = jnp.pad(emb_table, ((0, pad), (0, 0)))
    ec_pad = jnp.pad(electron_config, ((0, pad), (0, 0)))
    t2 = pl.pallas_call(
        _h_table_kernel,
        grid=(1,),
        in_specs=[
            full((128, EMBD)), full((128, 20)), full((20, EMBD)),
            full((1, EMBD)), full((B, 1)), full((B, 1)),
            full((OFF, EMBD)), full((OFF, EMBD)),
        ],
        out_specs=pl.BlockSpec((128, EMBD), lambda g: (0, 0)),
        out_shape=jax.ShapeDtypeStruct((128, EMBD), f32),
    )(emb_pad, ec_pad, cfg_W.T, cfg_b.reshape(1, EMBD), multiplicity,
      charge, mult_table, charge_table)
    cls_ids = MAX_Z + 1 + jnp.arange(B, dtype=jnp.int32)[:, None]
    az_idx = jnp.concatenate([cls_ids, atomic_numbers], axis=1)
    h = _sc_gather(t2, az_idx.reshape(B * M)).reshape(B, M, EMBD)
    return (h, e, msk)
